# baseline (speedup 1.0000x reference)
import jax
import jax.numpy as jnp
from jax.experimental import pallas as pl
from jax.experimental.pallas import tpu as pltpu

_BE = 256
_BU = 512


def _agg_kernel(head_ref, tail_ref, type_ref, ent_ref, w_ref,
                agg_ref, segmax_sc, den_sc):
    p = pl.program_id(0)
    b = pl.program_id(1)
    n_ent = ent_ref.shape[0]
    be = head_ref.shape[0]
    r = w_ref.shape[0]

    @pl.when((p == 0) & (b == 0))
    def _init():
        segmax_sc[...] = jnp.zeros_like(segmax_sc)
        den_sc[...] = jnp.zeros_like(den_sc)
        agg_ref[...] = jnp.zeros_like(agg_ref)

    iota_n = jax.lax.broadcasted_iota(jnp.int32, (be, n_ent), 1)
    mask_h = head_ref[...] == iota_n
    oh_h = mask_h.astype(jnp.float32)
    oh_t = (tail_ref[...] == iota_n).astype(jnp.float32)
    rel_oh = ((type_ref[...] - 1) ==
              jax.lax.broadcasted_iota(jnp.int32, (be, r), 1)).astype(jnp.float32)
    rel = jnp.dot(rel_oh, w_ref[...], preferred_element_type=jnp.float32, precision=jax.lax.Precision.HIGHEST)

    h_emb = jnp.dot(oh_h, ent_ref[...], preferred_element_type=jnp.float32, precision=jax.lax.Precision.HIGHEST)
    t_emb = jnp.dot(oh_t, ent_ref[...], preferred_element_type=jnp.float32, precision=jax.lax.Precision.HIGHEST)
    hr = h_emb * rel
    tr = t_emb * rel
    hn = jnp.sqrt(jnp.sum(hr * hr, axis=1, keepdims=True))
    tn = jnp.sqrt(jnp.sum(tr * tr, axis=1, keepdims=True))
    w = (hn * tn) ** 2

    @pl.when(p == 0)
    def _pass_max():
        blockmax = jnp.max(jnp.where(mask_h, w, 0.0), axis=0, keepdims=True)
        segmax_sc[...] = jnp.maximum(segmax_sc[...], blockmax)

    @pl.when(p == 1)
    def _pass_den():
        segmax_e = jnp.sum(jnp.where(mask_h, segmax_sc[...], 0.0),
                           axis=1, keepdims=True)
        num = jnp.exp(w - segmax_e)
        den_sc[...] = den_sc[...] + jnp.sum(jnp.where(mask_h, num, 0.0),
                                            axis=0, keepdims=True)

    @pl.when(p == 2)
    def _pass_scatter():
        segmax_e = jnp.sum(jnp.where(mask_h, segmax_sc[...], 0.0),
                           axis=1, keepdims=True)
        den_e = jnp.sum(jnp.where(mask_h, den_sc[...], 0.0),
                        axis=1, keepdims=True)
        sm = jnp.exp(w - segmax_e) / den_e
        contrib = jax.lax.dot_general(
            oh_h, sm * tr, (((0,), (0,)), ((), ())),
            preferred_element_type=jnp.float32, precision=jax.lax.Precision.HIGHEST)
        agg_ref[...] = agg_ref[...] + contrib


def _user_kernel(im_ref, ent_ref, uemb_ref, w_ref, out_ref):
    acc = jnp.dot(im_ref[...], ent_ref[...], preferred_element_type=jnp.float32, precision=jax.lax.Precision.HIGHEST)
    logits = jax.lax.dot_general(
        uemb_ref[...], w_ref[...], (((1,), (1,)), ((), ())),
        preferred_element_type=jnp.float32, precision=jax.lax.Precision.HIGHEST)
    score = jax.nn.softmax(logits, axis=-1)
    sw = jnp.dot(score, w_ref[...], preferred_element_type=jnp.float32, precision=jax.lax.Precision.HIGHEST)
    out_ref[...] = acc + sw * acc


@jax.jit
def kernel(entity_emb, user_emb, edge_index, edge_type, interact_mat, weight):
    n_ent, d = entity_emb.shape
    n_users = user_emb.shape[0]
    e = edge_index.shape[1]
    r = weight.shape[0]

    head = edge_index[0].reshape(e, 1)
    tail = edge_index[1].reshape(e, 1)
    etype = edge_type.reshape(e, 1)
    nb = e // _BE

    entity_agg = pl.pallas_call(
        _agg_kernel,
        grid=(3, nb),
        in_specs=[
            pl.BlockSpec((_BE, 1), lambda p, b: (b, 0)),
            pl.BlockSpec((_BE, 1), lambda p, b: (b, 0)),
            pl.BlockSpec((_BE, 1), lambda p, b: (b, 0)),
            pl.BlockSpec((n_ent, d), lambda p, b: (0, 0)),
            pl.BlockSpec((r, d), lambda p, b: (0, 0)),
        ],
        out_specs=pl.BlockSpec((n_ent, d), lambda p, b: (0, 0)),
        out_shape=jax.ShapeDtypeStruct((n_ent, d), jnp.float32),
        scratch_shapes=[
            pltpu.VMEM((1, n_ent), jnp.float32),
            pltpu.VMEM((1, n_ent), jnp.float32),
        ],
    )(head, tail, etype, entity_emb, weight)

    user_agg = pl.pallas_call(
        _user_kernel,
        grid=(n_users // _BU,),
        in_specs=[
            pl.BlockSpec((_BU, n_ent), lambda u: (u, 0)),
            pl.BlockSpec((n_ent, d), lambda u: (0, 0)),
            pl.BlockSpec((_BU, d), lambda u: (u, 0)),
            pl.BlockSpec((r, d), lambda u: (0, 0)),
        ],
        out_specs=pl.BlockSpec((_BU, d), lambda u: (u, 0)),
        out_shape=jax.ShapeDtypeStruct((n_users, d), jnp.float32),
    )(interact_mat, entity_emb, user_emb, weight)

    return (entity_agg, user_agg)
